# baseline (speedup 1.0000x reference)
import functools

import jax
import jax.numpy as jnp
from jax.experimental import pallas as pl
from jax.experimental.pallas import tpu as pltpu

N = 10000
E = 160000
DF = 256
DE = 16
H = 256
BD = 8
G = 16
EPS = 1e-5

BN = 1000
BE = 2000

_P = jax.lax.Precision.HIGHEST


def _dot(a, b):
    return jax.lax.dot(a, b, precision=_P, preferred_element_type=jnp.float32)


def _prologue_body(x_ref, fg_ref, Win_ref, bin_ref, A1_ref, Wux1_ref, bux1_ref,
                   gates_ref, nc1_ref, ox1_ref):
    g = jax.nn.sigmoid(fg_ref[...])
    gates_ref[...] = g
    h0 = jnp.maximum(_dot(x_ref[...] * g, Win_ref[...]) + bin_ref[...], 0.0)
    nc1_ref[...] = _dot(h0, A1_ref[...])
    ox1_ref[...] = _dot(h0, Wux1_ref[...]) + bux1_ref[...]


def _prologue(x, fg, Win, bin_, A1, Wux1, bux1):
    grid = N // BN
    const = lambda i: (0, 0)
    return pl.pallas_call(
        _prologue_body,
        grid=(grid,),
        in_specs=[
            pl.BlockSpec((BN, DF), lambda i: (i, 0)),
            pl.BlockSpec((1, DF), const),
            pl.BlockSpec((DF, BD), const),
            pl.BlockSpec((1, BD), const),
            pl.BlockSpec((BD, H), const),
            pl.BlockSpec((BD, H), const),
            pl.BlockSpec((1, H), const),
        ],
        out_specs=[
            pl.BlockSpec((1, DF), const),
            pl.BlockSpec((BN, H), lambda i: (i, 0)),
            pl.BlockSpec((BN, H), lambda i: (i, 0)),
        ],
        out_shape=[
            jax.ShapeDtypeStruct((1, DF), jnp.float32),
            jax.ShapeDtypeStruct((N, H), jnp.float32),
            jax.ShapeDtypeStruct((N, H), jnp.float32),
        ],
    )(x, fg, Win, bin_, A1, Wux1, bux1)


def _ec_body(ea_ref, B1_ref, d1_ref, B2_ref, d2_ref, ec1_ref, ec2_ref):
    ea = ea_ref[...]
    ec1_ref[...] = _dot(ea, B1_ref[...]) + d1_ref[...]
    ec2_ref[...] = _dot(ea, B2_ref[...]) + d2_ref[...]


def _ec(ea, B1, d1, B2, d2):
    grid = E // BE
    const = lambda i: (0, 0)
    return pl.pallas_call(
        _ec_body,
        grid=(grid,),
        in_specs=[
            pl.BlockSpec((BE, DE), lambda i: (i, 0)),
            pl.BlockSpec((DE, H), const),
            pl.BlockSpec((1, H), const),
            pl.BlockSpec((DE, H), const),
            pl.BlockSpec((1, H), const),
        ],
        out_specs=[
            pl.BlockSpec((BE, H), lambda i: (i, 0)),
            pl.BlockSpec((BE, H), lambda i: (i, 0)),
        ],
        out_shape=[
            jax.ShapeDtypeStruct((E, H), jnp.float32),
            jax.ShapeDtypeStruct((E, H), jnp.float32),
        ],
    )(ea, B1, d1, B2, d2)


def _edge_body(src_ref, dst_ref, nc_ref, ec_ref, aggr_ref):
    @pl.when(pl.program_id(0) == 0)
    def _init():
        aggr_ref[...] = jnp.zeros_like(aggr_ref)

    def body(i, _):
        s = src_ref[0, 0, i]
        d = dst_ref[0, 0, i]
        row = nc_ref[pl.ds(s, 1), :] + ec_ref[pl.ds(i, 1), :]
        msg = jnp.maximum(row, 0.0)
        aggr_ref[pl.ds(d, 1), :] = aggr_ref[pl.ds(d, 1), :] + msg
        return 0

    jax.lax.fori_loop(0, BE, body, 0)


def _edge_aggregate(src3, dst3, nc, ec):
    grid = E // BE
    return pl.pallas_call(
        _edge_body,
        grid=(grid,),
        in_specs=[
            pl.BlockSpec((1, 1, BE), lambda i: (i, 0, 0), memory_space=pltpu.SMEM),
            pl.BlockSpec((1, 1, BE), lambda i: (i, 0, 0), memory_space=pltpu.SMEM),
            pl.BlockSpec((N, H), lambda i: (0, 0)),
            pl.BlockSpec((BE, H), lambda i: (i, 0)),
        ],
        out_specs=pl.BlockSpec((N, H), lambda i: (0, 0)),
        out_shape=jax.ShapeDtypeStruct((N, H), jnp.float32),
    )(src3, dst3, nc, ec)


def _update1_body(ox_ref, ag_ref, Wga_ref, Wgb_ref, bg_ref, Wua_ref, Wub_ref,
                  bu_ref, s_ref, b_ref, A2_ref, Wux2_ref, bux2_ref,
                  nc2_ref, ox2_ref):
    ox = ox_ref[...]
    ag = ag_ref[...]
    gate = jax.nn.sigmoid(_dot(ox, Wga_ref[...]) + _dot(ag, Wgb_ref[...]) + bg_ref[...])
    upd = jnp.maximum(_dot(ox, Wua_ref[...]) + _dot(ag, Wub_ref[...]) + bu_ref[...], 0.0)
    h = gate * upd + (1.0 - gate) * ox
    h = jnp.maximum(h * s_ref[...] + b_ref[...], 0.0)
    nc2_ref[...] = _dot(h, A2_ref[...])
    ox2_ref[...] = _dot(h, Wux2_ref[...]) + bux2_ref[...]


def _update1(ox, ag, Wga, Wgb, bg, Wua, Wub, bu, s, b, A2, Wux2, bux2):
    grid = N // BN
    const = lambda i: (0, 0)
    wspec = pl.BlockSpec((H, H), const)
    vspec = pl.BlockSpec((1, H), const)
    return pl.pallas_call(
        _update1_body,
        grid=(grid,),
        in_specs=[
            pl.BlockSpec((BN, H), lambda i: (i, 0)),
            pl.BlockSpec((BN, H), lambda i: (i, 0)),
            wspec, wspec, vspec, wspec, wspec, vspec, vspec, vspec,
            wspec, wspec, vspec,
        ],
        out_specs=[
            pl.BlockSpec((BN, H), lambda i: (i, 0)),
            pl.BlockSpec((BN, H), lambda i: (i, 0)),
        ],
        out_shape=[
            jax.ShapeDtypeStruct((N, H), jnp.float32),
            jax.ShapeDtypeStruct((N, H), jnp.float32),
        ],
    )(ox, ag, Wga, Wgb, bg, Wua, Wub, bu, s, b, A2, Wux2, bux2)


def _update2_body(ox_ref, ag_ref, batch_ref, Wga_ref, Wgb_ref, bg_ref,
                  Wua_ref, Wub_ref, bu_ref, s_ref, b_ref, sums_ref):
    ox = ox_ref[...]
    ag = ag_ref[...]
    gate = jax.nn.sigmoid(_dot(ox, Wga_ref[...]) + _dot(ag, Wgb_ref[...]) + bg_ref[...])
    upd = jnp.maximum(_dot(ox, Wua_ref[...]) + _dot(ag, Wub_ref[...]) + bu_ref[...], 0.0)
    h = gate * upd + (1.0 - gate) * ox
    h = jnp.maximum(h * s_ref[...] + b_ref[...], 0.0)
    seg = jax.lax.broadcasted_iota(jnp.int32, (BN, G), 1)
    oh = (batch_ref[...] == seg).astype(jnp.float32)
    part = jax.lax.dot_general(oh, h, (((0,), (0,)), ((), ())),
                               precision=_P, preferred_element_type=jnp.float32)

    @pl.when(pl.program_id(0) == 0)
    def _init():
        sums_ref[...] = jnp.zeros_like(sums_ref)

    sums_ref[...] += part


def _update2(ox, ag, batch2, Wga, Wgb, bg, Wua, Wub, bu, s, b):
    grid = N // BN
    const = lambda i: (0, 0)
    wspec = pl.BlockSpec((H, H), const)
    vspec = pl.BlockSpec((1, H), const)
    return pl.pallas_call(
        _update2_body,
        grid=(grid,),
        in_specs=[
            pl.BlockSpec((BN, H), lambda i: (i, 0)),
            pl.BlockSpec((BN, H), lambda i: (i, 0)),
            pl.BlockSpec((BN, 1), lambda i: (i, 0)),
            wspec, wspec, vspec, wspec, wspec, vspec, vspec, vspec,
        ],
        out_specs=pl.BlockSpec((G, H), const),
        out_shape=jax.ShapeDtypeStruct((G, H), jnp.float32),
    )(ox, ag, batch2, Wga, Wgb, bg, Wua, Wub, bu, s, b)


def _head_body(sums_ref, batch_ref, Wfc_ref, bfc_ref, WcT_ref, bc_ref, out_ref):
    seg = jax.lax.broadcasted_iota(jnp.int32, (N, G), 1)
    oh = (batch_ref[...] == seg).astype(jnp.float32)
    counts = jnp.sum(oh, axis=0)[:, None]
    emb = sums_ref[...] / jnp.maximum(counts, 1.0)
    emb = jnp.maximum(_dot(emb, Wfc_ref[...]) + bfc_ref[...], 0.0)
    out_ref[...] = jnp.sum(emb * WcT_ref[...], axis=1, keepdims=True) + bc_ref[0, 0]


def _head(sums, batch2, Wfc, bfc, WcT, bc):
    return pl.pallas_call(
        _head_body,
        in_specs=[
            pl.BlockSpec((G, H), lambda: (0, 0)),
            pl.BlockSpec((N, 1), lambda: (0, 0)),
            pl.BlockSpec((H, H // 2), lambda: (0, 0)),
            pl.BlockSpec((1, H // 2), lambda: (0, 0)),
            pl.BlockSpec((1, H // 2), lambda: (0, 0)),
            pl.BlockSpec((1, 1), lambda: (0, 0), memory_space=pltpu.SMEM),
        ],
        out_specs=pl.BlockSpec((G, 1), lambda: (0, 0)),
        out_shape=jax.ShapeDtypeStruct((G, 1), jnp.float32),
    )(sums, batch2, Wfc, bfc, WcT, bc)


def kernel(x, edge_index, edge_attr, batch, feature_gates, Win, bin_,
           c1_Wsn, c1_bsn, c1_We, c1_be, c1_Wmsg, c1_bmsg, c1_Wux, c1_bux,
           c1_Wum, c1_bum, c1_Wg, c1_bg, bn1_g, bn1_b,
           c2_Wsn, c2_bsn, c2_We, c2_be, c2_Wmsg, c2_bmsg, c2_Wux, c2_bux,
           c2_Wum, c2_bum, c2_Wg, c2_bg, bn2_g, bn2_b, Wfc, bfc, Wc, bc):
    A1 = c1_Wsn @ c1_Wmsg[:H]
    B1 = c1_We @ c1_Wmsg[H:]
    d1 = (c1_bsn @ c1_Wmsg[:H] + c1_be @ c1_Wmsg[H:] + c1_bmsg)[None, :]
    A2 = c2_Wsn @ c2_Wmsg[:H]
    B2 = c2_We @ c2_Wmsg[H:]
    d2 = (c2_bsn @ c2_Wmsg[:H] + c2_be @ c2_Wmsg[H:] + c2_bmsg)[None, :]
    bn_s1 = (bn1_g / jnp.sqrt(1.0 + EPS))[None, :]
    bn_s2 = (bn2_g / jnp.sqrt(1.0 + EPS))[None, :]

    src3 = edge_index[0].reshape(E // BE, 1, BE)
    dst3 = edge_index[1].reshape(E // BE, 1, BE)
    batch2 = batch.reshape(N, 1)

    gates2, nc1, ox1 = _prologue(x, feature_gates[None, :], Win, bin_[None, :],
                                 A1, c1_Wux, c1_bux[None, :])
    ec1, ec2 = _ec(edge_attr, B1, d1, B2, d2)

    aggr1 = _edge_aggregate(src3, dst3, nc1, ec1)
    nc2, ox2 = _update1(ox1, aggr1, c1_Wg[:H], c1_Wg[H:], c1_bg[None, :],
                        c1_Wum[:H], c1_Wum[H:], c1_bum[None, :],
                        bn_s1, bn1_b[None, :], A2, c2_Wux, c2_bux[None, :])

    aggr2 = _edge_aggregate(src3, dst3, nc2, ec2)
    sums = _update2(ox2, aggr2, batch2, c2_Wg[:H], c2_Wg[H:], c2_bg[None, :],
                    c2_Wum[:H], c2_Wum[H:], c2_bum[None, :],
                    bn_s2, bn2_b[None, :])

    pred = _head(sums, batch2, Wfc, bfc[None, :], Wc[:, 0][None, :],
                 bc.reshape(1, 1))
    return pred.reshape(G), gates2.reshape(DF)
